# Initial kernel scaffold; baseline (speedup 1.0000x reference)
import jax
import jax.numpy as jnp
from jax.experimental import pallas as pl


def kernel(center, pos, neg, center_table, neigh_table):
    raise NotImplementedError("write your pallas kernel here")


# speedup vs baseline: 3.6356x; 3.6356x over previous
import functools

import jax
import jax.numpy as jnp
from jax import lax
from jax.experimental import pallas as pl
from jax.experimental.pallas import tpu as pltpu
from jax.experimental.pallas import tpu_sc as plsc

B = 16384
D = 64
NEGS = 20
NC = 2
NS = 16
NW = NC * NS
BPW = B // NW
CB = 64
NCHUNK = BPW // CB
NROWS = CB * NEGS
IDXW = 128


def _sc_body(center_hbm, pos_hbm, negf_hbm, ctab_hbm, ntab_hbm, out_hbm,
             cidx, pidx, nidx, crow, prow, nrow, scores, sem_c, sem_p, sem_n):
    wid = lax.axis_index("s") * NC + lax.axis_index("c")
    base = wid * BPW

    def chunk_body(c, carry):
        cb0 = base + c * CB
        pltpu.sync_copy(center_hbm.at[pl.ds(cb0, CB)], cidx)
        pltpu.sync_copy(pos_hbm.at[pl.ds(cb0, CB)], pidx)
        pltpu.sync_copy(negf_hbm.at[pl.ds(cb0 * NEGS, NROWS)], nidx)

        copies = [
            pltpu.async_copy(ctab_hbm.at[cidx], crow, sem_c),
            pltpu.async_copy(ntab_hbm.at[pidx], prow, sem_p),
        ]
        for r in range(NROWS // IDXW):
            copies.append(pltpu.async_copy(
                ntab_hbm.at[nidx.at[pl.ds(r * IDXW, IDXW)]],
                nrow.at[pl.ds(r * IDXW, IDXW)], sem_n))
        for cp in copies:
            cp.wait()

        for g in range(CB // 16):
            lanes = lax.iota(jnp.int32, 16) + g * 16
            off = c * CB + g * 16
            nlanes = [lanes * NEGS + n for n in range(NEGS)]

            dv0 = jnp.zeros((16,), jnp.int32)
            cT = plsc.load_gather(crow, [lanes, dv0])
            scores[pl.ds(off, 16)] = cT * plsc.load_gather(prow, [lanes, dv0])
            for n in range(NEGS):
                scores[pl.ds((n + 1) * BPW + off, 16)] = (
                    cT * plsc.load_gather(nrow, [nlanes[n], dv0]))

            def d_body(d, _):
                dv = jnp.full((16,), d, jnp.int32)
                cTd = plsc.load_gather(crow, [lanes, dv])
                plsc.addupdate(scores.at[pl.ds(off, 16)],
                               cTd * plsc.load_gather(prow, [lanes, dv]))
                for n in range(NEGS):
                    plsc.addupdate(
                        scores.at[pl.ds((n + 1) * BPW + off, 16)],
                        cTd * plsc.load_gather(nrow, [nlanes[n], dv]))
                return 0

            lax.fori_loop(1, D, d_body, 0)
        return carry

    lax.fori_loop(0, NCHUNK, chunk_body, 0)
    for j in range(NEGS + 1):
        pltpu.sync_copy(scores.at[pl.ds(j * BPW, BPW)],
                        out_hbm.at[pl.ds(j * B + base, BPW)])


_sc_scores = functools.partial(
    pl.kernel,
    out_type=jax.ShapeDtypeStruct(((NEGS + 1) * B,), jnp.float32),
    mesh=plsc.VectorSubcoreMesh(core_axis_name="c", subcore_axis_name="s"),
    compiler_params=pltpu.CompilerParams(
        needs_layout_passes=False, use_tc_tiling_on_sc=False),
    scratch_types=[
        pltpu.VMEM((CB,), jnp.int32),
        pltpu.VMEM((CB,), jnp.int32),
        pltpu.VMEM((NROWS,), jnp.int32),
        pltpu.VMEM((CB, D), jnp.float32),
        pltpu.VMEM((CB, D), jnp.float32),
        pltpu.VMEM((NROWS, D), jnp.float32),
        pltpu.VMEM(((NEGS + 1) * BPW,), jnp.float32),
        pltpu.SemaphoreType.DMA,
        pltpu.SemaphoreType.DMA,
        pltpu.SemaphoreType.DMA,
    ],
)(_sc_body)


def _tc_loss(x_ref, o_ref):
    x = jnp.clip(x_ref[...], -10.0, 10.0)
    ls = -jnp.log1p(jnp.exp(-x))
    o_ref[0, 0] = -jnp.sum(ls) / B


def kernel(center, pos, neg, center_table, neigh_table):
    center = center.astype(jnp.int32)
    pos = pos.astype(jnp.int32)
    negf = neg.astype(jnp.int32).reshape(B * NEGS)
    scores = _sc_scores(center, pos, negf, center_table, neigh_table)
    flat = scores.reshape((NEGS + 1) * B // 128, 128)
    loss = pl.pallas_call(
        _tc_loss,
        out_shape=jax.ShapeDtypeStruct((1, 1), jnp.float32),
        out_specs=pl.BlockSpec(memory_space=pltpu.SMEM),
    )(flat)
    return loss[0, 0]


# speedup vs baseline: 4.0645x; 1.1180x over previous
import functools

import jax
import jax.numpy as jnp
from jax import lax
from jax.experimental import pallas as pl
from jax.experimental.pallas import tpu as pltpu
from jax.experimental.pallas import tpu_sc as plsc

B = 16384
D = 64
NEGS = 20
NC = 2
NS = 16
NW = NC * NS
BPW = B // NW
CB = 64
NCHUNK = BPW // CB
NROWS = CB * NEGS
IDXW = 128


def _sc_body(center_hbm, pos_hbm, negf_hbm, ctab_hbm, ntab_hbm, out_hbm,
             cidx, pidx, nidx, crow, prow, nrow, scores, sem_c, sem_p, sem_n):
    wid = lax.axis_index("s") * NC + lax.axis_index("c")
    base = wid * BPW

    def chunk_body(c, carry):
        cb0 = base + c * CB
        pltpu.sync_copy(center_hbm.at[pl.ds(cb0, CB)], cidx)
        pltpu.sync_copy(pos_hbm.at[pl.ds(cb0, CB)], pidx)
        pltpu.sync_copy(negf_hbm.at[pl.ds(cb0 * NEGS, NROWS)], nidx)

        copies = [
            pltpu.async_copy(ctab_hbm.at[cidx], crow, sem_c),
            pltpu.async_copy(ntab_hbm.at[pidx], prow, sem_p),
        ]
        for r in range(NROWS // IDXW):
            copies.append(pltpu.async_copy(
                ntab_hbm.at[nidx.at[pl.ds(r * IDXW, IDXW)]],
                nrow.at[pl.ds(r * IDXW, IDXW)], sem_n))
        for cp in copies:
            cp.wait()

        def group_body(g, carry2):
            lanes = lax.iota(jnp.int32, 16) + g * 16
            off = c * CB + g * 16
            nlanes = [lanes * NEGS + n for n in range(NEGS)]

            accs = None
            for d in range(D):
                dv = jnp.full((16,), d, jnp.int32)
                cT = plsc.load_gather(crow, [lanes, dv])
                pT = plsc.load_gather(prow, [lanes, dv])
                nT = [plsc.load_gather(nrow, [nlanes[n], dv])
                      for n in range(NEGS)]
                if accs is None:
                    accs = [cT * pT] + [cT * x for x in nT]
                else:
                    accs[0] = accs[0] + cT * pT
                    for n in range(NEGS):
                        accs[n + 1] = accs[n + 1] + cT * nT[n]

            scores[pl.ds(off, 16)] = accs[0]
            for n in range(NEGS):
                scores[pl.ds((n + 1) * BPW + off, 16)] = accs[n + 1]
            return carry2

        lax.fori_loop(0, CB // 16, group_body, 0)
        return carry

    lax.fori_loop(0, NCHUNK, chunk_body, 0)
    for j in range(NEGS + 1):
        pltpu.sync_copy(scores.at[pl.ds(j * BPW, BPW)],
                        out_hbm.at[pl.ds(j * B + base, BPW)])


_sc_scores = functools.partial(
    pl.kernel,
    out_type=jax.ShapeDtypeStruct(((NEGS + 1) * B,), jnp.float32),
    mesh=plsc.VectorSubcoreMesh(core_axis_name="c", subcore_axis_name="s"),
    compiler_params=pltpu.CompilerParams(
        needs_layout_passes=False, use_tc_tiling_on_sc=False),
    scratch_types=[
        pltpu.VMEM((CB,), jnp.int32),
        pltpu.VMEM((CB,), jnp.int32),
        pltpu.VMEM((NROWS,), jnp.int32),
        pltpu.VMEM((CB, D), jnp.float32),
        pltpu.VMEM((CB, D), jnp.float32),
        pltpu.VMEM((NROWS, D), jnp.float32),
        pltpu.VMEM(((NEGS + 1) * BPW,), jnp.float32),
        pltpu.SemaphoreType.DMA,
        pltpu.SemaphoreType.DMA,
        pltpu.SemaphoreType.DMA,
    ],
)(_sc_body)


def _tc_loss(x_ref, o_ref):
    x = jnp.clip(x_ref[...], -10.0, 10.0)
    ls = -jnp.log1p(jnp.exp(-x))
    o_ref[0, 0] = -jnp.sum(ls) / B


def kernel(center, pos, neg, center_table, neigh_table):
    center = center.astype(jnp.int32)
    pos = pos.astype(jnp.int32)
    negf = neg.astype(jnp.int32).reshape(B * NEGS)
    scores = _sc_scores(center, pos, negf, center_table, neigh_table)
    flat = scores.reshape((NEGS + 1) * B // 128, 128)
    loss = pl.pallas_call(
        _tc_loss,
        out_shape=jax.ShapeDtypeStruct((1, 1), jnp.float32),
        out_specs=pl.BlockSpec(memory_space=pltpu.SMEM),
    )(flat)
    return loss[0, 0]


# speedup vs baseline: 5.0414x; 1.2403x over previous
import functools

import jax
import jax.numpy as jnp
from jax import lax
from jax.experimental import pallas as pl
from jax.experimental.pallas import tpu as pltpu
from jax.experimental.pallas import tpu_sc as plsc

B = 16384
D = 64
NEGS = 20
NJ = NEGS + 1
NC = 2
NS = 16
NW = NC * NS
BPW = B // NW
CB = 32
NCHUNK = BPW // CB
XROWS = CB * NJ
IDXW = 112


def _sc_body(center_hbm, pn_hbm, ctab_hbm, ntab_hbm, out_hbm,
             cidx, xidx, crow, xrow, scores, sem_c, sem_x):
    wid = lax.axis_index("s") * NC + lax.axis_index("c")
    base = wid * BPW

    def chunk_body(c, carry):
        cb0 = base + c * CB
        pltpu.sync_copy(center_hbm.at[pl.ds(cb0, CB)], cidx)
        pltpu.sync_copy(pn_hbm.at[pl.ds(cb0 * NJ, XROWS)], xidx)

        copies = [pltpu.async_copy(ctab_hbm.at[cidx], crow, sem_c)]
        for r in range(XROWS // IDXW):
            copies.append(pltpu.async_copy(
                ntab_hbm.at[xidx.at[pl.ds(r * IDXW, IDXW)]],
                xrow.at[pl.ds(r * IDXW, IDXW)], sem_x))
        for cp in copies:
            cp.wait()

        def group_body(g, carry2):
            lanes = lax.iota(jnp.int32, 16) + g * 16
            off = c * CB + g * 16
            rot = lax.iota(jnp.int32, 16)
            xlanes = [lanes * NJ + j for j in range(NJ)]

            def d_block(k, accs):
                dv0 = rot + k * 16
                accs = list(accs)
                for t in range(16):
                    dv = (dv0 + t) & (D - 1)
                    cT = plsc.load_gather(crow, [lanes, dv])
                    for j in range(NJ):
                        accs[j] = accs[j] + cT * plsc.load_gather(
                            xrow, [xlanes[j], dv])
                return tuple(accs)

            zero = jnp.zeros((16,), jnp.float32)
            accs = lax.fori_loop(0, D // 16, d_block,
                                 tuple(zero for _ in range(NJ)))
            for j in range(NJ):
                scores[pl.ds(j * BPW + off, 16)] = accs[j]
            return carry2

        lax.fori_loop(0, CB // 16, group_body, 0)
        return carry

    lax.fori_loop(0, NCHUNK, chunk_body, 0)
    for j in range(NJ):
        pltpu.sync_copy(scores.at[pl.ds(j * BPW, BPW)],
                        out_hbm.at[pl.ds(j * B + base, BPW)])


_sc_scores = functools.partial(
    pl.kernel,
    out_type=jax.ShapeDtypeStruct((NJ * B,), jnp.float32),
    mesh=plsc.VectorSubcoreMesh(core_axis_name="c", subcore_axis_name="s"),
    compiler_params=pltpu.CompilerParams(
        needs_layout_passes=False, use_tc_tiling_on_sc=False),
    scratch_types=[
        pltpu.VMEM((CB,), jnp.int32),
        pltpu.VMEM((XROWS,), jnp.int32),
        pltpu.VMEM((CB, D), jnp.float32),
        pltpu.VMEM((XROWS, D), jnp.float32),
        pltpu.VMEM((NJ * BPW,), jnp.float32),
        pltpu.SemaphoreType.DMA,
        pltpu.SemaphoreType.DMA,
    ],
)(_sc_body)


def _tc_loss(x_ref, o_ref):
    x = jnp.clip(x_ref[...], -10.0, 10.0)
    ls = -jnp.log1p(jnp.exp(-x))
    o_ref[0, 0] = -jnp.sum(ls) / B


def kernel(center, pos, neg, center_table, neigh_table):
    center = center.astype(jnp.int32)
    pn = jnp.concatenate(
        [pos.astype(jnp.int32)[:, None], neg.astype(jnp.int32)], axis=1)
    pn = pn.reshape(B * NJ)
    scores = _sc_scores(center, pn, center_table, neigh_table)
    flat = scores.reshape(NJ * B // 128, 128)
    loss = pl.pallas_call(
        _tc_loss,
        out_shape=jax.ShapeDtypeStruct((1, 1), jnp.float32),
        out_specs=pl.BlockSpec(memory_space=pltpu.SMEM),
    )(flat)
    return loss[0, 0]
